# baseline (speedup 1.0000x reference)
import jax
import jax.numpy as jnp
from jax.experimental import pallas as pl
from jax.experimental.pallas import tpu as pltpu

_N = 3584
_KEEP = 7 * 512
_K = int(0.6 * _N)
_BR = 256
_STEPS = _N // _BR


def _loss_kernel(x_ref, out_ref, dw_ref, colabs_ref):
    i = pl.program_id(0)
    x = x_ref[...]
    rs = jnp.sum(jnp.sum(x, axis=0), axis=1, keepdims=True)
    dw_ref[pl.ds(i * _BR, _BR), :] = rs
    part = jnp.sum(jnp.abs(x), axis=1)

    @pl.when(i == 0)
    def _():
        colabs_ref[0:7, :] = part

    @pl.when(i > 0)
    def _():
        colabs_ref[0:7, :] = colabs_ref[0:7, :] + part

    @pl.when(i == _STEPS - 1)
    def _():
        adw_r = jnp.transpose(jnp.abs(dw_ref[...]))
        idx_r = jax.lax.broadcasted_iota(jnp.int32, (1, _N), 1)

        def body(c, rank):
            a_c = jnp.abs(dw_ref[pl.ds(c * _BR, _BR), :])
            i_c = (jax.lax.broadcasted_iota(jnp.int32, (_BR, 1), 0)
                   + c * _BR)
            less = (a_c < adw_r).astype(jnp.float32)
            tie = ((a_c == adw_r) & (i_c < idx_r)).astype(jnp.float32)
            return rank + jnp.sum(less + tie, axis=0, keepdims=True)

        rank = jax.lax.fori_loop(0, _STEPS, body,
                                 jnp.zeros((1, _N), jnp.float32))
        mask = (rank < float(_K)).astype(jnp.float32)
        total = jnp.zeros((1, 1), jnp.float32)
        for j in range(7):
            total = total + jnp.sum(
                mask[:, j * 512:(j + 1) * 512] * colabs_ref[j:j + 1, :],
                keepdims=True)
        out_ref[...] = total / (_N * _KEEP)


def kernel(ref_latents, iters):
    xt = jnp.transpose(ref_latents, (1, 0, 2))
    loss = pl.pallas_call(
        _loss_kernel,
        grid=(_STEPS,),
        in_specs=[pl.BlockSpec((7, _BR, 512), lambda i: (0, i, 0))],
        out_specs=pl.BlockSpec((1, 1), lambda i: (0, 0)),
        out_shape=jax.ShapeDtypeStruct((1, 1), jnp.float32),
        scratch_shapes=[pltpu.VMEM((_N, 1), jnp.float32),
                        pltpu.VMEM((8, 512), jnp.float32)],
    )(xt)
    rw = jnp.maximum(0.0, (iters - 50) / (300 - 50))
    return rw * loss[0, 0]


# speedup vs baseline: 1.0903x; 1.0903x over previous
import jax
import jax.numpy as jnp
from jax.experimental import pallas as pl
from jax.experimental.pallas import tpu as pltpu

_N = 3584
_KEEP = 7 * 512
_K = int(0.6 * _N)
_BR = 512
_STEPS = _N // _BR
_HB = _BR // 2


def _loss_kernel(xa_ref, xb_ref, out_ref, dw_ref, colabs_ref):
    i = pl.program_id(0)
    xa = xa_ref[...]
    xb = xb_ref[...]
    rsa = jnp.sum(jnp.sum(xa, axis=0), axis=1, keepdims=True)
    rsb = jnp.sum(jnp.sum(xb, axis=0), axis=1, keepdims=True)
    dw_ref[pl.ds(i * _BR, _HB), :] = rsa
    dw_ref[pl.ds(i * _BR + _HB, _HB), :] = rsb
    part = jnp.sum(jnp.abs(xa), axis=1) + jnp.sum(jnp.abs(xb), axis=1)

    @pl.when(i == 0)
    def _():
        colabs_ref[0:7, :] = part

    @pl.when(i > 0)
    def _():
        colabs_ref[0:7, :] = colabs_ref[0:7, :] + part

    @pl.when(i == _STEPS - 1)
    def _():
        adw_r = jnp.transpose(jnp.abs(dw_ref[...]))
        idx_r = jax.lax.broadcasted_iota(jnp.int32, (1, _N), 1)

        def body(c, rank):
            a_c = jnp.abs(dw_ref[pl.ds(c * _BR, _BR), :])
            i_c = (jax.lax.broadcasted_iota(jnp.int32, (_BR, 1), 0)
                   + c * _BR)
            less = (a_c < adw_r).astype(jnp.float32)
            tie = ((a_c == adw_r) & (i_c < idx_r)).astype(jnp.float32)
            return rank + jnp.sum(less + tie, axis=0, keepdims=True)

        rank = jax.lax.fori_loop(0, _STEPS, body,
                                 jnp.zeros((1, _N), jnp.float32))
        mask = (rank < float(_K)).astype(jnp.float32)
        total = jnp.zeros((1, 1), jnp.float32)
        for j in range(7):
            total = total + jnp.sum(
                mask[:, j * 512:(j + 1) * 512] * colabs_ref[j:j + 1, :],
                keepdims=True)
        out_ref[...] = total / (_N * _KEEP)


def kernel(ref_latents, iters):
    xt = jnp.transpose(ref_latents, (1, 0, 2))
    loss = pl.pallas_call(
        _loss_kernel,
        grid=(_STEPS,),
        in_specs=[pl.BlockSpec((7, _HB, 512), lambda i: (0, 2 * i, 0)),
                  pl.BlockSpec((7, _HB, 512), lambda i: (0, 2 * i + 1, 0))],
        out_specs=pl.BlockSpec((1, 1), lambda i: (0, 0)),
        out_shape=jax.ShapeDtypeStruct((1, 1), jnp.float32),
        scratch_shapes=[pltpu.VMEM((_N, 1), jnp.float32),
                        pltpu.VMEM((8, 512), jnp.float32)],
    )(xt, xt)
    rw = jnp.maximum(0.0, (iters - 50) / (300 - 50))
    return rw * loss[0, 0]
